# Initial kernel scaffold; baseline (speedup 1.0000x reference)
import functools

import jax
import jax.numpy as jnp
from jax import lax
from jax.experimental import pallas as pl
from jax.experimental.pallas import tpu as pltpu
from jax.experimental.pallas import tpu_sc as plsc

N = 10000
D = 256
H = 8
DH = 32
E = 160000
NTILES = 16
EPT = E // NTILES
BATCH = 125
NB = EPT // BATCH
ROWPT = N // NTILES
ACCW = 144
SCALE = 1.0 / (DH ** 0.5)


def _qkv_body(x_ref, wq_ref, wk_ref, wv_ref, q0, q1, k0, k1, v0, v1):
    x = x_ref[...]
    dn = (((1,), (1,)), ((), ()))
    q = lax.dot_general(x, wq_ref[...], dn, preferred_element_type=jnp.float32)
    k = lax.dot_general(x, wk_ref[...], dn, preferred_element_type=jnp.float32)
    v = lax.dot_general(x, wv_ref[...], dn, preferred_element_type=jnp.float32)
    q = q * SCALE
    q0[...] = q[:, :128]
    q1[...] = q[:, 128:]
    k0[...] = k[:, :128]
    k1[...] = k[:, 128:]
    v0[...] = v[:, :128]
    v1[...] = v[:, 128:]


def _qkv(x, Wq, Wk, Wv):
    blk = 1000
    grid = (N // blk,)
    half = jax.ShapeDtypeStruct((N, 128), jnp.float32)
    return pl.pallas_call(
        _qkv_body,
        grid=grid,
        in_specs=[
            pl.BlockSpec((blk, D), lambda i: (i, 0)),
            pl.BlockSpec((D, D), lambda i: (0, 0)),
            pl.BlockSpec((D, D), lambda i: (0, 0)),
            pl.BlockSpec((D, D), lambda i: (0, 0)),
        ],
        out_specs=[pl.BlockSpec((blk, 128), lambda i: (i, 0))] * 6,
        out_shape=[half] * 6,
    )(x, Wq, Wk, Wv)


def _sc_edge_call(q0, q1, k0, k1, v0, v1, ei, ew, we_exp):
    mesh = plsc.VectorSubcoreMesh(core_axis_name="c", subcore_axis_name="s")

    @functools.partial(
        pl.kernel,
        mesh=mesh,
        out_type=jax.ShapeDtypeStruct((2, N, ACCW), jnp.float32),
        scratch_types=[
            pltpu.VMEM((NB, BATCH), jnp.int32),
            pltpu.VMEM((NB, BATCH), jnp.int32),
            pltpu.VMEM((NB, BATCH), jnp.float32),
            pltpu.VMEM((BATCH, 128), jnp.float32),
            pltpu.VMEM((BATCH, 128), jnp.float32),
            pltpu.VMEM((BATCH, 128), jnp.float32),
            pltpu.VMEM((BATCH, ACCW), jnp.float32),
            pltpu.VMEM((16,), jnp.float32),
            pltpu.VMEM((16,), jnp.float32),
            pltpu.VMEM_SHARED((N, ACCW), jnp.float32),
            pltpu.SemaphoreType.DMA,
            pltpu.SemaphoreType.DMA,
            pltpu.SemaphoreType.DMA,
        ],
    )
    def sc_kernel(q0h, q1h, k0h, k1h, v0h, v1h, ei_h, ew_h, we_h, out_h,
                  src_v, tgt_v, ew_v, qv, kv, vv, stage, pbuf, webuf, acc,
                  sem_q, sem_k, sem_v):
        cid = lax.axis_index("c")
        sid = lax.axis_index("s")

        pltpu.sync_copy(ei_h.at[0, sid], src_v)
        pltpu.sync_copy(ei_h.at[1, sid], tgt_v)
        pltpu.sync_copy(ew_h.at[sid], ew_v)
        pltpu.sync_copy(we_h.at[cid], webuf)

        zero = jnp.zeros((16,), jnp.float32)

        def zero_row(r, carry):
            for i in range(ACCW // 16):
                stage[r, pl.ds(16 * i, 16)] = zero
            return carry

        lax.fori_loop(0, BATCH, zero_row, 0)
        for piece in range(ROWPT // BATCH):
            pltpu.sync_copy(stage, acc.at[pl.ds(sid * ROWPT + piece * BATCH, BATCH)])
        plsc.subcore_barrier()

        ar = lax.iota(jnp.int32, 16)
        lane_masks = [(ar == h).astype(jnp.float32) for h in range(4)]
        mask4 = (ar < 4).astype(jnp.float32)
        wevec = webuf[...]

        def run_half(qh, kh, vh):
            def batch_body(j, carry):
                cq = pltpu.async_copy(qh.at[src_v.at[j]], qv, sem_q)
                ck = pltpu.async_copy(kh.at[tgt_v.at[j]], kv, sem_k)
                cv = pltpu.async_copy(vh.at[tgt_v.at[j]], vv, sem_v)
                cq.wait()
                ck.wait()
                cv.wait()

                def edge_body(e, icarry):
                    svec = zero
                    for h in range(4):
                        prod = (qv[e, pl.ds(32 * h, 16)] * kv[e, pl.ds(32 * h, 16)]
                                + qv[e, pl.ds(32 * h + 16, 16)] * kv[e, pl.ds(32 * h + 16, 16)])
                        svec = svec + jnp.sum(prod) * lane_masks[h]
                    ewe = ew_v[j, e]
                    pvec = jnp.exp(svec + ewe * wevec)
                    pbuf[...] = pvec
                    stage[e, pl.ds(128, 16)] = pvec * mask4
                    for h in range(4):
                        ph = pbuf[h]
                        stage[e, pl.ds(32 * h, 16)] = vv[e, pl.ds(32 * h, 16)] * ph
                        stage[e, pl.ds(32 * h + 16, 16)] = vv[e, pl.ds(32 * h + 16, 16)] * ph
                    return icarry

                lax.fori_loop(0, BATCH, edge_body, 0)
                pltpu.sync_copy(stage, acc.at[src_v.at[j]], add=True)
                return carry

            lax.fori_loop(0, NB, batch_body, 0)

        @pl.when(cid == 0)
        def _():
            run_half(q0h, k0h, v0h)

        @pl.when(cid == 1)
        def _():
            run_half(q1h, k1h, v1h)

        plsc.subcore_barrier()
        pltpu.sync_copy(acc.at[pl.ds(sid * ROWPT, ROWPT)],
                        out_h.at[cid, pl.ds(sid * ROWPT, ROWPT)])

    return sc_kernel(q0, q1, k0, k1, v0, v1, ei, ew, we_exp)


def _combine_body(a0_ref, a1_ref, x_ref, wo_ref, g_ref, b_ref, s_ref, o_ref):
    a0 = a0_ref[...]
    a1 = a1_ref[...]
    S = s_ref[...]
    dn = (((1,), (0,)), ((), ()))
    de0 = lax.dot_general(a0[:, 128:132], S, dn, preferred_element_type=jnp.float32)
    de1 = lax.dot_general(a1[:, 128:132], S, dn, preferred_element_type=jnp.float32)
    de0 = jnp.where(de0 > 0.0, de0, 1.0)
    de1 = jnp.where(de1 > 0.0, de1, 1.0)
    attn = jnp.concatenate([a0[:, :128] / de0, a1[:, :128] / de1], axis=1)
    dnt = (((1,), (1,)), ((), ()))
    out = lax.dot_general(attn, wo_ref[...], dnt, preferred_element_type=jnp.float32)
    y = out + x_ref[...]
    mean = jnp.mean(y, axis=1, keepdims=True)
    var = jnp.mean((y - mean) ** 2, axis=1, keepdims=True)
    o_ref[...] = (y - mean) * lax.rsqrt(var + 1e-5) * g_ref[...] + b_ref[...]


def _combine(a0, a1, x, Wo, gamma, beta, S):
    blk = 1000
    grid = (N // blk,)
    return pl.pallas_call(
        _combine_body,
        grid=grid,
        in_specs=[
            pl.BlockSpec((blk, ACCW), lambda i: (i, 0)),
            pl.BlockSpec((blk, ACCW), lambda i: (i, 0)),
            pl.BlockSpec((blk, D), lambda i: (i, 0)),
            pl.BlockSpec((D, D), lambda i: (0, 0)),
            pl.BlockSpec((1, D), lambda i: (0, 0)),
            pl.BlockSpec((1, D), lambda i: (0, 0)),
            pl.BlockSpec((4, 128), lambda i: (0, 0)),
        ],
        out_specs=pl.BlockSpec((blk, D), lambda i: (i, 0)),
        out_shape=jax.ShapeDtypeStruct((N, D), jnp.float32),
    )(a0, a1, x, Wo, gamma, beta, S)


def kernel(node_embeddings, edge_index, edge_weights, Wq, Wk, Wv, We, Wo, gamma, beta):
    x = node_embeddings[0]
    q0, q1, k0, k1, v0, v1 = _qkv(x, Wq, Wk, Wv)

    ei = edge_index.astype(jnp.int32).reshape(2, NTILES, NB, BATCH)
    ew = edge_weights.reshape(NTILES, NB, BATCH)
    we_flat = We[:, 0]
    we_exp = jnp.zeros((2, 16), jnp.float32)
    we_exp = we_exp.at[0, :4].set(we_flat[:4]).at[1, :4].set(we_flat[4:])

    acc = _sc_edge_call(q0, q1, k0, k1, v0, v1, ei, ew, we_exp)

    S = (jnp.arange(128)[None, :] // DH == jnp.arange(4)[:, None]).astype(jnp.float32)
    ln = _combine(acc[0], acc[1], x, Wo, gamma, beta, S)
    return ln[None]


# speedup vs baseline: 6.5190x; 6.5190x over previous
import functools

import jax
import jax.numpy as jnp
from jax import lax
from jax.experimental import pallas as pl
from jax.experimental.pallas import tpu as pltpu
from jax.experimental.pallas import tpu_sc as plsc

N = 10000
D = 256
H = 8
DH = 32
E = 160000
NTILES = 16
EPT = E // NTILES
BATCH = 80
NB = EPT // BATCH
NG = BATCH // 16
ROWPT = N // NTILES
SCALE = 1.0 / (DH ** 0.5)


def _qkv_body(x_ref, wq_ref, wk_ref, wv_ref, q0, q1, k0, k1, v0, v1):
    x = x_ref[...]
    dn = (((1,), (1,)), ((), ()))
    q = lax.dot_general(x, wq_ref[...], dn, preferred_element_type=jnp.float32)
    k = lax.dot_general(x, wk_ref[...], dn, preferred_element_type=jnp.float32)
    v = lax.dot_general(x, wv_ref[...], dn, preferred_element_type=jnp.float32)
    q = q * SCALE
    q0[...] = q[:, :128]
    q1[...] = q[:, 128:]
    k0[...] = k[:, :128]
    k1[...] = k[:, 128:]
    v0[...] = v[:, :128]
    v1[...] = v[:, 128:]


def _qkv(x, Wq, Wk, Wv):
    blk = 1000
    grid = (N // blk,)
    half = jax.ShapeDtypeStruct((N, 128), jnp.float32)
    return pl.pallas_call(
        _qkv_body,
        grid=grid,
        in_specs=[
            pl.BlockSpec((blk, D), lambda i: (i, 0)),
            pl.BlockSpec((D, D), lambda i: (0, 0)),
            pl.BlockSpec((D, D), lambda i: (0, 0)),
            pl.BlockSpec((D, D), lambda i: (0, 0)),
        ],
        out_specs=[pl.BlockSpec((blk, 128), lambda i: (i, 0))] * 6,
        out_shape=[half] * 6,
    )(x, Wq, Wk, Wv)


def _sc_edge_call(q0, q1, k0, k1, v0, v1, ei, ew, we_exp):
    mesh = plsc.VectorSubcoreMesh(core_axis_name="c", subcore_axis_name="s")

    @functools.partial(
        pl.kernel,
        mesh=mesh,
        compiler_params=pltpu.CompilerParams(
            use_tc_tiling_on_sc=False, needs_layout_passes=False),
        out_type=(
            jax.ShapeDtypeStruct((2, N, 128), jnp.float32),
            jax.ShapeDtypeStruct((2, N, 16), jnp.float32),
        ),
        scratch_types=[
            pltpu.VMEM((1, BATCH), jnp.int32),
            pltpu.VMEM((1, BATCH), jnp.int32),
            pltpu.VMEM((1, BATCH), jnp.float32),
            pltpu.VMEM((BATCH, 128), jnp.float32),
            pltpu.VMEM((BATCH, 128), jnp.float32),
            pltpu.VMEM((BATCH, 128), jnp.float32),
            pltpu.VMEM((BATCH, 16), jnp.float32),
            pltpu.VMEM((4, 16), jnp.float32),
            pltpu.VMEM_SHARED((N, 128), jnp.float32),
            pltpu.VMEM_SHARED((N, 16), jnp.float32),
            pltpu.SemaphoreType.DMA,
            pltpu.SemaphoreType.DMA,
            pltpu.SemaphoreType.DMA,
        ],
    )
    def sc_kernel(q0h, q1h, k0h, k1h, v0h, v1h, ei_h, ew_h, we_h,
                  outn_h, outd_h,
                  srcb, tgtb, ewb, qv, kv, vv, stden, webuf,
                  accn, accd, sem_q, sem_k, sem_v):
        cid = lax.axis_index("c")
        sid = lax.axis_index("s")

        pltpu.sync_copy(we_h.at[cid], webuf)

        zero = jnp.zeros((16,), jnp.float32)

        def zero_bufs(r, carry):
            for i in range(8):
                qv[r, pl.ds(16 * i, 16)] = zero
            stden[r, pl.ds(0, 16)] = zero
            return carry

        lax.fori_loop(0, BATCH, zero_bufs, 0)

        def zero_acc(k, carry):
            pltpu.sync_copy(qv, accn.at[pl.ds(sid * ROWPT + BATCH * k, BATCH)])
            pltpu.sync_copy(stden, accd.at[pl.ds(sid * ROWPT + BATCH * k, BATCH)])
            return carry

        lax.fori_loop(0, ROWPT // BATCH, zero_acc, 0)
        rem = ROWPT - (ROWPT // BATCH) * BATCH
        pltpu.sync_copy(qv.at[pl.ds(0, rem)],
                        accn.at[pl.ds(sid * ROWPT + ROWPT - rem, rem)])
        pltpu.sync_copy(stden.at[pl.ds(0, rem)],
                        accd.at[pl.ds(sid * ROWPT + ROWPT - rem, rem)])
        plsc.subcore_barrier()

        def run_half(qh, kh, vh):
            iota16 = lax.iota(jnp.int32, 16)
            step16 = jnp.full((16,), 16, jnp.int32)
            zidx = jnp.zeros((16,), jnp.int32)
            wrows = [webuf[hh, pl.ds(0, 16)] for hh in range(4)]

            def batch_body(j, carry):
                pltpu.sync_copy(ei_h.at[0, sid, j], srcb.at[0])
                pltpu.sync_copy(ei_h.at[1, sid, j], tgtb.at[0])
                pltpu.sync_copy(ew_h.at[sid, j], ewb.at[0])
                cq = pltpu.async_copy(qh.at[srcb.at[0]], qv, sem_q)
                ck = pltpu.async_copy(kh.at[tgtb.at[0]], kv, sem_k)
                cv = pltpu.async_copy(vh.at[tgtb.at[0]], vv, sem_v)
                cq.wait()
                ck.wait()
                cv.wait()

                for g in range(NG):
                    eidx = iota16 + jnp.full((16,), 16 * g, jnp.int32)
                    ewg = plsc.load_gather(ewb, [zidx, eidx])
                    for h in range(4):
                        acc = zero
                        for d in range(DH):
                            col = jnp.full((16,), 32 * h + d, jnp.int32)
                            qc = plsc.load_gather(qv, [eidx, col])
                            kc = plsc.load_gather(kv, [eidx, col])
                            acc = acc + qc * kc
                        ph = jnp.exp(acc + ewg * wrows[h])
                        plsc.store_scatter(stden, [eidx, jnp.full((16,), h, jnp.int32)], ph)
                        for d in range(DH):
                            col = jnp.full((16,), 32 * h + d, jnp.int32)
                            vc = plsc.load_gather(vv, [eidx, col])
                            plsc.store_scatter(vv, [eidx, col], vc * ph)
                pltpu.sync_copy(vv, accn.at[srcb.at[0]], add=True)
                pltpu.sync_copy(stden, accd.at[srcb.at[0]], add=True)
                return carry

            lax.fori_loop(0, NB, batch_body, 0)

        @pl.when(cid == 0)
        def _():
            run_half(q0h, k0h, v0h)

        @pl.when(cid == 1)
        def _():
            run_half(q1h, k1h, v1h)

        plsc.subcore_barrier()
        pltpu.sync_copy(accn.at[pl.ds(sid * ROWPT, ROWPT)],
                        outn_h.at[cid, pl.ds(sid * ROWPT, ROWPT)])
        pltpu.sync_copy(accd.at[pl.ds(sid * ROWPT, ROWPT)],
                        outd_h.at[cid, pl.ds(sid * ROWPT, ROWPT)])

    return sc_kernel(q0, q1, k0, k1, v0, v1, ei, ew, we_exp)


def _combine_body(n0_ref, n1_ref, d0_ref, d1_ref, x_ref, wo_ref, g_ref, b_ref,
                  s_ref, o_ref):
    S = s_ref[...]
    dn = (((1,), (0,)), ((), ()))
    de0 = lax.dot_general(d0_ref[...][:, :4], S, dn, preferred_element_type=jnp.float32)
    de1 = lax.dot_general(d1_ref[...][:, :4], S, dn, preferred_element_type=jnp.float32)
    de0 = jnp.where(de0 > 0.0, de0, 1.0)
    de1 = jnp.where(de1 > 0.0, de1, 1.0)
    attn = jnp.concatenate([n0_ref[...] / de0, n1_ref[...] / de1], axis=1)
    dnt = (((1,), (1,)), ((), ()))
    out = lax.dot_general(attn, wo_ref[...], dnt, preferred_element_type=jnp.float32)
    y = out + x_ref[...]
    mean = jnp.mean(y, axis=1, keepdims=True)
    var = jnp.mean((y - mean) ** 2, axis=1, keepdims=True)
    o_ref[...] = (y - mean) * lax.rsqrt(var + 1e-5) * g_ref[...] + b_ref[...]


def _combine(n0, n1, d0, d1, x, Wo, gamma, beta, S):
    blk = 1000
    grid = (N // blk,)
    return pl.pallas_call(
        _combine_body,
        grid=grid,
        in_specs=[
            pl.BlockSpec((blk, 128), lambda i: (i, 0)),
            pl.BlockSpec((blk, 128), lambda i: (i, 0)),
            pl.BlockSpec((blk, 16), lambda i: (i, 0)),
            pl.BlockSpec((blk, 16), lambda i: (i, 0)),
            pl.BlockSpec((blk, D), lambda i: (i, 0)),
            pl.BlockSpec((D, D), lambda i: (0, 0)),
            pl.BlockSpec((1, D), lambda i: (0, 0)),
            pl.BlockSpec((1, D), lambda i: (0, 0)),
            pl.BlockSpec((4, 128), lambda i: (0, 0)),
        ],
        out_specs=pl.BlockSpec((blk, D), lambda i: (i, 0)),
        out_shape=jax.ShapeDtypeStruct((N, D), jnp.float32),
    )(n0, n1, d0, d1, x, Wo, gamma, beta, S)


def kernel(node_embeddings, edge_index, edge_weights, Wq, Wk, Wv, We, Wo, gamma, beta):
    x = node_embeddings[0]
    q0, q1, k0, k1, v0, v1 = _qkv(x, Wq, Wk, Wv)

    ei = edge_index.astype(jnp.int32).reshape(2, NTILES, NB, BATCH)
    ew = edge_weights.reshape(NTILES, NB, BATCH)
    we_exp = jnp.broadcast_to(We[:, 0].reshape(2, 4, 1), (2, 4, 16)).astype(jnp.float32)

    outn, outd = _sc_edge_call(q0, q1, k0, k1, v0, v1, ei, ew, we_exp)

    S = (jnp.arange(128)[None, :] // DH == jnp.arange(4)[:, None]).astype(jnp.float32)
    ln = _combine(outn[0], outn[1], outd[0], outd[1], x, Wo,
                  gamma.reshape(1, D), beta.reshape(1, D), S)
    return ln[None]


# speedup vs baseline: 7.1411x; 1.0954x over previous
import functools

import jax
import jax.numpy as jnp
from jax import lax
from jax.experimental import pallas as pl
from jax.experimental.pallas import tpu as pltpu
from jax.experimental.pallas import tpu_sc as plsc

N = 10000
NPAD = 10008
D = 256
H = 8
DH = 32
E = 160000
NTILES = 16
BATCH = 32
NB = 314
EPTP = NB * BATCH
NG = BATCH // 16
ROWPT = N // NTILES
SCALE = 1.0 / (DH ** 0.5)


def _qkv_body(x_ref, wq_ref, wk_ref, wv_ref, qs, kvs):
    x = x_ref[...]
    dn = (((1,), (1,)), ((), ()))
    q = lax.dot_general(x, wq_ref[...], dn, preferred_element_type=jnp.float32)
    k = lax.dot_general(x, wk_ref[...], dn, preferred_element_type=jnp.float32)
    v = lax.dot_general(x, wv_ref[...], dn, preferred_element_type=jnp.float32)
    q = q * SCALE
    qs[0] = q[:, :128]
    qs[1] = q[:, 128:]
    kvs[0] = jnp.concatenate([k[:, :128], v[:, :128]], axis=1)
    kvs[1] = jnp.concatenate([k[:, 128:], v[:, 128:]], axis=1)


def _qkv(x, Wq, Wk, Wv):
    blk = 1000
    grid = (N // blk,)
    return pl.pallas_call(
        _qkv_body,
        grid=grid,
        in_specs=[
            pl.BlockSpec((blk, D), lambda i: (i, 0)),
            pl.BlockSpec((D, D), lambda i: (0, 0)),
            pl.BlockSpec((D, D), lambda i: (0, 0)),
            pl.BlockSpec((D, D), lambda i: (0, 0)),
        ],
        out_specs=[
            pl.BlockSpec((2, blk, 128), lambda i: (0, i, 0)),
            pl.BlockSpec((2, blk, 256), lambda i: (0, i, 0)),
        ],
        out_shape=[
            jax.ShapeDtypeStruct((2, N, 128), jnp.float32),
            jax.ShapeDtypeStruct((2, N, 256), jnp.float32),
        ],
    )(x, Wq, Wk, Wv)


def _sc_edge_call(qs, kvs, edata, we_exp):
    mesh = plsc.VectorSubcoreMesh(core_axis_name="c", subcore_axis_name="s")

    vb = lambda shape: pltpu.VMEM(shape, jnp.float32)
    ib = lambda shape: pltpu.VMEM(shape, jnp.int32)

    @functools.partial(
        pl.kernel,
        mesh=mesh,
        compiler_params=pltpu.CompilerParams(
            use_tc_tiling_on_sc=False, needs_layout_passes=False),
        out_type=(
            jax.ShapeDtypeStruct((2, N, 128), jnp.float32),
            jax.ShapeDtypeStruct((2, N, 8), jnp.float32),
        ),
        scratch_types=[
            ib((4, BATCH)), ib((4, BATCH)),
            ib((2, BATCH)), ib((2, BATCH)),
            vb((BATCH, 128)), vb((BATCH, 128)),
            vb((BATCH, 256)), vb((BATCH, 256)),
            vb((BATCH, 128)), vb((BATCH, 128)),
            vb((BATCH, 8)), vb((BATCH, 8)),
            vb((4, 16)),
            pltpu.VMEM_SHARED((NPAD, 128), jnp.float32),
            pltpu.VMEM_SHARED((NPAD, 8), jnp.float32),
        ] + [pltpu.SemaphoreType.DMA] * 10,
    )
    def sc_kernel(qs_h, kvs_h, ed_h, we_h, zn_h, zd_h,
                  outn_h, outd_h,
                  edidxA, edidxB, sidxA, sidxB, qvA, qvB, kvA, kvB,
                  vvA, vvB, sdA, sdB, webuf, accn, accd,
                  semIA, semIB, semGqA, semGqB, semGkA, semGkB,
                  semSnA, semSnB, semSdA, semSdB):
        cid = lax.axis_index("c")
        sid = lax.axis_index("s")

        edidx = [edidxA, edidxB]
        sidx = [sidxA, sidxB]
        qv = [qvA, qvB]
        kvb = [kvA, kvB]
        vvst = [vvA, vvB]
        stden = [sdA, sdB]
        semI = [semIA, semIB]
        semGq = [semGqA, semGqB]
        semGk = [semGkA, semGkB]
        semSn = [semSnA, semSnB]
        semSd = [semSdA, semSdB]

        pltpu.sync_copy(we_h.at[cid], webuf)

        zero = jnp.zeros((16,), jnp.float32)

        pltpu.sync_copy(zn_h.at[pl.ds(0, ROWPT)],
                        accn.at[pl.ds(sid * ROWPT, ROWPT)])
        pltpu.sync_copy(zd_h.at[pl.ds(0, ROWPT)],
                        accd.at[pl.ds(sid * ROWPT, ROWPT)])

        @pl.when(sid == NTILES - 1)
        def _():
            pltpu.sync_copy(zn_h.at[pl.ds(0, NPAD - N)], accn.at[pl.ds(N, NPAD - N)])
            pltpu.sync_copy(zd_h.at[pl.ds(0, NPAD - N)], accd.at[pl.ds(N, NPAD - N)])

        iota0 = lax.iota(jnp.int32, 16)
        rloc = iota0 // 4
        cloc = (iota0 % 4) + jnp.full((16,), 4, jnp.int32)
        for c in range(BATCH // 4):
            rowv = rloc + jnp.full((16,), 4 * c, jnp.int32)
            plsc.store_scatter(sdA, [rowv, cloc], zero)
            plsc.store_scatter(sdB, [rowv, cloc], zero)

        plsc.subcore_barrier()

        def run_half(qh, kvh):
            iota16 = lax.iota(jnp.int32, 16)
            wrows = [webuf[hh, pl.ds(0, 16)] for hh in range(4)]

            def snapshot(p):
                for g in range(NG):
                    sidx[p][0, pl.ds(16 * g, 16)] = edidx[p][3, pl.ds(16 * g, 16)]
                    sidx[p][1, pl.ds(16 * g, 16)] = edidx[p][2, pl.ds(16 * g, 16)]

            one = jnp.full((16,), 1, jnp.int32)
            c128 = jnp.full((16,), 128, jnp.int32)

            def compute(p):
                for g in range(NG):
                    eidx = iota16 + jnp.full((16,), 16 * g, jnp.int32)
                    ewg = plsc.bitcast(
                        plsc.load_gather(sidx[p], [one, eidx]),
                        jnp.float32)
                    for h in range(4):
                        col0 = jnp.full((16,), 32 * h, jnp.int32)

                        def dot_body(d, carry):
                            acc, col = carry
                            qc = plsc.load_gather(qv[p], [eidx, col])
                            kc = plsc.load_gather(kvb[p], [eidx, col])
                            return acc + qc * kc, col + one

                        acc, _ = lax.fori_loop(0, DH, dot_body, (zero, col0),
                                               unroll=8)
                        ph = jnp.exp(acc + ewg * wrows[h])
                        plsc.store_scatter(
                            stden[p], [eidx, jnp.full((16,), h, jnp.int32)], ph)

                        def v_body(d, col):
                            vc = plsc.load_gather(kvb[p], [eidx, col + c128])
                            plsc.store_scatter(vvst[p], [eidx, col], vc * ph)
                            return col + one

                        lax.fori_loop(0, DH, v_body, col0, unroll=8)

            pltpu.sync_copy(ed_h.at[sid, 0], edidxA)
            pltpu.sync_copy(ed_h.at[sid, 1], edidxB)
            pltpu.async_copy(qh.at[edidxA.at[0]], qvA, semGqA)
            pltpu.async_copy(kvh.at[edidxA.at[1]], kvA, semGkA)

            def body(i, carry):
                for k in range(2):
                    j = 2 * i + k
                    p, q = k, 1 - k
                    pltpu.make_async_copy(qh.at[edidx[p].at[0]], qv[p], semGq[p]).wait()
                    pltpu.make_async_copy(kvh.at[edidx[p].at[1]], kvb[p], semGk[p]).wait()

                    @pl.when(i > 0)
                    def _():
                        pltpu.make_async_copy(
                            qh.at[pl.ds(0, BATCH)], vvst[p], semSn[p]).wait()
                        pltpu.make_async_copy(
                            outd_h.at[cid, pl.ds(0, BATCH)], stden[p], semSd[p]).wait()

                    snapshot(p)
                    jn = jnp.minimum(j + 2, NB - 1)
                    pltpu.async_copy(ed_h.at[sid, jn], edidx[p], semI[p])

                    if k == 0:
                        @pl.when(i > 0)
                        def _():
                            pltpu.make_async_copy(
                                ed_h.at[sid, 0], edidx[q], semI[q]).wait()

                        pltpu.async_copy(qh.at[edidx[q].at[0]], qv[q], semGq[q])
                        pltpu.async_copy(kvh.at[edidx[q].at[1]], kvb[q], semGk[q])
                    else:
                        @pl.when(i < NB // 2 - 1)
                        def _():
                            pltpu.make_async_copy(
                                ed_h.at[sid, 0], edidx[q], semI[q]).wait()
                            pltpu.async_copy(qh.at[edidx[q].at[0]], qv[q], semGq[q])
                            pltpu.async_copy(kvh.at[edidx[q].at[1]], kvb[q], semGk[q])

                    compute(p)
                    pltpu.async_copy(vvst[p], accn.at[sidx[p].at[0]], semSn[p], add=True)
                    pltpu.async_copy(stden[p], accd.at[sidx[p].at[0]], semSd[p], add=True)
                return carry

            lax.fori_loop(0, NB // 2, body, 0)

            for p in range(2):
                pltpu.make_async_copy(qh.at[pl.ds(0, BATCH)], vvst[p], semSn[p]).wait()
                pltpu.make_async_copy(
                    outd_h.at[cid, pl.ds(0, BATCH)], stden[p], semSd[p]).wait()
            pltpu.make_async_copy(ed_h.at[sid, 0], edidxB, semIB).wait()
            pltpu.make_async_copy(ed_h.at[sid, 0], edidxA, semIA).wait()

        run_half(qs_h.at[cid], kvs_h.at[cid])

        plsc.subcore_barrier()
        pltpu.sync_copy(accn.at[pl.ds(sid * ROWPT, ROWPT)],
                        outn_h.at[cid, pl.ds(sid * ROWPT, ROWPT)])
        pltpu.sync_copy(accd.at[pl.ds(sid * ROWPT, ROWPT)],
                        outd_h.at[cid, pl.ds(sid * ROWPT, ROWPT)])

    zn = jnp.zeros((ROWPT + 8, 128), jnp.float32)
    zd = jnp.zeros((ROWPT + 8, 8), jnp.float32)
    return sc_kernel(qs, kvs, edata, we_exp, zn, zd)


def _combine_body(n0_ref, n1_ref, d0_ref, d1_ref, x_ref, wo_ref, g_ref, b_ref,
                  s_ref, o_ref):
    S = s_ref[...]
    dn = (((1,), (0,)), ((), ()))
    de0 = lax.dot_general(d0_ref[...][:, :4], S, dn, preferred_element_type=jnp.float32)
    de1 = lax.dot_general(d1_ref[...][:, :4], S, dn, preferred_element_type=jnp.float32)
    de0 = jnp.where(de0 > 0.0, de0, 1.0)
    de1 = jnp.where(de1 > 0.0, de1, 1.0)
    attn = jnp.concatenate([n0_ref[...] / de0, n1_ref[...] / de1], axis=1)
    dnt = (((1,), (1,)), ((), ()))
    out = lax.dot_general(attn, wo_ref[...], dnt, preferred_element_type=jnp.float32)
    y = out + x_ref[...]
    mean = jnp.mean(y, axis=1, keepdims=True)
    var = jnp.mean((y - mean) ** 2, axis=1, keepdims=True)
    o_ref[...] = (y - mean) * lax.rsqrt(var + 1e-5) * g_ref[...] + b_ref[...]


def _combine(n0, n1, d0, d1, x, Wo, gamma, beta, S):
    blk = 1000
    grid = (N // blk,)
    return pl.pallas_call(
        _combine_body,
        grid=grid,
        in_specs=[
            pl.BlockSpec((blk, 128), lambda i: (i, 0)),
            pl.BlockSpec((blk, 128), lambda i: (i, 0)),
            pl.BlockSpec((blk, 8), lambda i: (i, 0)),
            pl.BlockSpec((blk, 8), lambda i: (i, 0)),
            pl.BlockSpec((blk, D), lambda i: (i, 0)),
            pl.BlockSpec((D, D), lambda i: (0, 0)),
            pl.BlockSpec((1, D), lambda i: (0, 0)),
            pl.BlockSpec((1, D), lambda i: (0, 0)),
            pl.BlockSpec((4, 128), lambda i: (0, 0)),
        ],
        out_specs=pl.BlockSpec((blk, D), lambda i: (i, 0)),
        out_shape=jax.ShapeDtypeStruct((N, D), jnp.float32),
    )(n0, n1, d0, d1, x, Wo, gamma, beta, S)


def kernel(node_embeddings, edge_index, edge_weights, Wq, Wk, Wv, We, Wo, gamma, beta):
    x = node_embeddings[0]
    qs, kvs = _qkv(x, Wq, Wk, Wv)

    pad = EPTP - E // NTILES
    ei = edge_index.astype(jnp.int32)
    src = ei[0].reshape(NTILES, E // NTILES)
    tgt = ei[1].reshape(NTILES, E // NTILES)
    ewb = lax.bitcast_convert_type(edge_weights, jnp.int32).reshape(NTILES, -1)
    zpad = jnp.zeros((NTILES, pad), jnp.int32)
    srcg = jnp.concatenate([src, zpad], 1).reshape(NTILES, NB, BATCH)
    tgtp = jnp.concatenate([tgt, zpad], 1).reshape(NTILES, NB, BATCH)
    ewbp = jnp.concatenate([ewb, zpad], 1).reshape(NTILES, NB, BATCH)
    srcs = jnp.concatenate([src, jnp.full((NTILES, pad), N, jnp.int32)], 1)
    srcs = srcs.reshape(NTILES, NB, BATCH)
    edata = jnp.stack([srcg, tgtp, ewbp, srcs], axis=2)

    we_exp = jnp.broadcast_to(We[:, 0].reshape(2, 4, 1), (2, 4, 16)).astype(jnp.float32)

    outn, outd = _sc_edge_call(qs, kvs, edata, we_exp)

    S = (jnp.arange(128)[None, :] // DH == jnp.arange(4)[:, None]).astype(jnp.float32)
    ln = _combine(outn[0], outn[1], outd[0], outd[1], x, Wo,
                  gamma.reshape(1, D), beta.reshape(1, D), S)
    return ln[None]


# speedup vs baseline: 8.1642x; 1.1433x over previous
import functools

import jax
import jax.numpy as jnp
from jax import lax
from jax.experimental import pallas as pl
from jax.experimental.pallas import tpu as pltpu
from jax.experimental.pallas import tpu_sc as plsc

N = 10000
NPAD = 10008
D = 256
H = 8
DH = 32
E = 160000
NTILES = 16
BATCH = 16
NB = 628
EPTP = NB * BATCH
NG = BATCH // 16
ROWPT = N // NTILES
ACCW = 144
SCALE = 1.0 / (DH ** 0.5)


def _qkv_body(x_ref, wq_ref, wk_ref, wv_ref, qs, kvs):
    x = x_ref[...]
    dn = (((1,), (1,)), ((), ()))
    q = lax.dot_general(x, wq_ref[...], dn, preferred_element_type=jnp.float32)
    k = lax.dot_general(x, wk_ref[...], dn, preferred_element_type=jnp.float32)
    v = lax.dot_general(x, wv_ref[...], dn, preferred_element_type=jnp.float32)
    q = q * SCALE
    qs[0] = q[:, :128]
    qs[1] = q[:, 128:]
    kvs[0] = jnp.concatenate([k[:, :128], v[:, :128]], axis=1)
    kvs[1] = jnp.concatenate([k[:, 128:], v[:, 128:]], axis=1)


def _qkv(x, Wq, Wk, Wv):
    blk = 1000
    grid = (N // blk,)
    return pl.pallas_call(
        _qkv_body,
        grid=grid,
        in_specs=[
            pl.BlockSpec((blk, D), lambda i: (i, 0)),
            pl.BlockSpec((D, D), lambda i: (0, 0)),
            pl.BlockSpec((D, D), lambda i: (0, 0)),
            pl.BlockSpec((D, D), lambda i: (0, 0)),
        ],
        out_specs=[
            pl.BlockSpec((2, blk, 128), lambda i: (0, i, 0)),
            pl.BlockSpec((2, blk, 256), lambda i: (0, i, 0)),
        ],
        out_shape=[
            jax.ShapeDtypeStruct((2, N, 128), jnp.float32),
            jax.ShapeDtypeStruct((2, N, 256), jnp.float32),
        ],
    )(x, Wq, Wk, Wv)


def _sc_edge_call(qs, kvs, edata, we_exp):
    mesh = plsc.VectorSubcoreMesh(core_axis_name="c", subcore_axis_name="s")

    vb = lambda shape: pltpu.VMEM(shape, jnp.float32)
    ib = lambda shape: pltpu.VMEM(shape, jnp.int32)

    @functools.partial(
        pl.kernel,
        mesh=mesh,
        compiler_params=pltpu.CompilerParams(
            use_tc_tiling_on_sc=False, needs_layout_passes=False),
        out_type=jax.ShapeDtypeStruct((2, N, ACCW), jnp.float32),
        scratch_types=[
            ib((4, BATCH)), ib((4, BATCH)),
            ib((2, BATCH)), ib((2, BATCH)),
            vb((BATCH, 128)), vb((BATCH, 128)),
            vb((BATCH, 256)), vb((BATCH, 256)),
            vb((BATCH, ACCW)), vb((BATCH, ACCW)),
            vb((4, 16)),
            pltpu.VMEM_SHARED((NPAD, ACCW), jnp.float32),
        ] + [pltpu.SemaphoreType.DMA] * 8,
    )
    def sc_kernel(qs_h, kvs_h, ed_h, we_h, zn_h,
                  outn_h,
                  edidxA, edidxB, sidxA, sidxB, qvA, qvB, kvA, kvB,
                  vvA, vvB, webuf, accn,
                  semIA, semIB, semGqA, semGqB, semGkA, semGkB,
                  semSnA, semSnB):
        cid = lax.axis_index("c")
        sid = lax.axis_index("s")

        edidx = [edidxA, edidxB]
        sidx = [sidxA, sidxB]
        qv = [qvA, qvB]
        kvb = [kvA, kvB]
        vvst = [vvA, vvB]
        semI = [semIA, semIB]
        semGq = [semGqA, semGqB]
        semGk = [semGkA, semGkB]
        semSn = [semSnA, semSnB]

        pltpu.sync_copy(we_h.at[cid], webuf)

        zero = jnp.zeros((16,), jnp.float32)

        pltpu.sync_copy(zn_h.at[pl.ds(0, ROWPT)],
                        accn.at[pl.ds(sid * ROWPT, ROWPT)])

        @pl.when(sid == NTILES - 1)
        def _():
            pltpu.sync_copy(zn_h.at[pl.ds(0, NPAD - N)], accn.at[pl.ds(N, NPAD - N)])

        iota0 = lax.iota(jnp.int32, 16)
        for col in range(132, ACCW):
            colv = jnp.full((16,), col, jnp.int32)
            for g in range(NG):
                rowv = iota0 + jnp.full((16,), 16 * g, jnp.int32)
                plsc.store_scatter(vvA, [rowv, colv], zero)
                plsc.store_scatter(vvB, [rowv, colv], zero)

        plsc.subcore_barrier()

        def run_half(qh, kvh):
            iota16 = lax.iota(jnp.int32, 16)
            wrows = [webuf[hh, pl.ds(0, 16)] for hh in range(4)]

            def snapshot(p):
                for g in range(NG):
                    sidx[p][0, pl.ds(16 * g, 16)] = edidx[p][3, pl.ds(16 * g, 16)]
                    sidx[p][1, pl.ds(16 * g, 16)] = edidx[p][2, pl.ds(16 * g, 16)]

            one = jnp.full((16,), 1, jnp.int32)
            c128 = jnp.full((16,), 128, jnp.int32)

            def compute(p):
                for g in range(NG):
                    eidx = iota16 + jnp.full((16,), 16 * g, jnp.int32)
                    ewg = plsc.bitcast(
                        plsc.load_gather(sidx[p], [one, eidx]),
                        jnp.float32)
                    for h in range(4):
                        col0 = jnp.full((16,), 32 * h, jnp.int32)

                        def dot_body(d, carry):
                            acc, col = carry
                            qc = plsc.load_gather(qv[p], [eidx, col])
                            kc = plsc.load_gather(kvb[p], [eidx, col])
                            return acc + qc * kc, col + one

                        acc, _ = lax.fori_loop(0, DH, dot_body, (zero, col0),
                                               unroll=8)
                        ph = jnp.exp(acc + ewg * wrows[h])
                        plsc.store_scatter(
                            vvst[p], [eidx, jnp.full((16,), 128 + h, jnp.int32)], ph)

                        def v_body(d, col):
                            vc = plsc.load_gather(kvb[p], [eidx, col + c128])
                            plsc.store_scatter(vvst[p], [eidx, col], vc * ph)
                            return col + one

                        lax.fori_loop(0, DH, v_body, col0, unroll=8)

            pltpu.sync_copy(ed_h.at[sid, 0], edidxA)
            pltpu.sync_copy(ed_h.at[sid, 1], edidxB)
            pltpu.async_copy(qh.at[edidxA.at[0]], qvA, semGqA)
            pltpu.async_copy(kvh.at[edidxA.at[1]], kvA, semGkA)

            def body(i, carry):
                for k in range(2):
                    j = 2 * i + k
                    p, q = k, 1 - k
                    pltpu.make_async_copy(qh.at[edidx[p].at[0]], qv[p], semGq[p]).wait()
                    pltpu.make_async_copy(kvh.at[edidx[p].at[1]], kvb[p], semGk[p]).wait()

                    @pl.when(i > 0)
                    def _():
                        pltpu.make_async_copy(
                            outn_h.at[cid, pl.ds(0, BATCH)], vvst[p], semSn[p]).wait()

                    snapshot(p)
                    jn = jnp.minimum(j + 2, NB - 1)
                    pltpu.async_copy(ed_h.at[sid, jn], edidx[p], semI[p])

                    if k == 0:
                        @pl.when(i > 0)
                        def _():
                            pltpu.make_async_copy(
                                ed_h.at[sid, 0], edidx[q], semI[q]).wait()

                        pltpu.async_copy(qh.at[edidx[q].at[0]], qv[q], semGq[q])
                        pltpu.async_copy(kvh.at[edidx[q].at[1]], kvb[q], semGk[q])
                    else:
                        @pl.when(i < NB // 2 - 1)
                        def _():
                            pltpu.make_async_copy(
                                ed_h.at[sid, 0], edidx[q], semI[q]).wait()
                            pltpu.async_copy(qh.at[edidx[q].at[0]], qv[q], semGq[q])
                            pltpu.async_copy(kvh.at[edidx[q].at[1]], kvb[q], semGk[q])

                    compute(p)
                    pltpu.async_copy(vvst[p], accn.at[sidx[p].at[0]], semSn[p], add=True)
                return carry

            lax.fori_loop(0, NB // 2, body, 0)

            for p in range(2):
                pltpu.make_async_copy(
                    outn_h.at[cid, pl.ds(0, BATCH)], vvst[p], semSn[p]).wait()
            pltpu.make_async_copy(ed_h.at[sid, 0], edidxB, semIB).wait()
            pltpu.make_async_copy(ed_h.at[sid, 0], edidxA, semIA).wait()

        run_half(qs_h.at[cid], kvs_h.at[cid])

        plsc.subcore_barrier()
        pltpu.sync_copy(accn.at[pl.ds(sid * ROWPT, ROWPT)],
                        outn_h.at[cid, pl.ds(sid * ROWPT, ROWPT)])

    zn = jnp.zeros((ROWPT + 8, ACCW), jnp.float32)
    return sc_kernel(qs, kvs, edata, we_exp, zn)


def _combine_body(a0_ref, a1_ref, x_ref, wo_ref, g_ref, b_ref, s_ref, o_ref):
    a0 = a0_ref[...]
    a1 = a1_ref[...]
    S = s_ref[...]
    dn = (((1,), (0,)), ((), ()))
    de0 = lax.dot_general(a0[:, 128:132], S, dn, preferred_element_type=jnp.float32)
    de1 = lax.dot_general(a1[:, 128:132], S, dn, preferred_element_type=jnp.float32)
    de0 = jnp.where(de0 > 0.0, de0, 1.0)
    de1 = jnp.where(de1 > 0.0, de1, 1.0)
    attn = jnp.concatenate([a0[:, :128] / de0, a1[:, :128] / de1], axis=1)
    dnt = (((1,), (1,)), ((), ()))
    out = lax.dot_general(attn, wo_ref[...], dnt, preferred_element_type=jnp.float32)
    y = out + x_ref[...]
    mean = jnp.mean(y, axis=1, keepdims=True)
    var = jnp.mean((y - mean) ** 2, axis=1, keepdims=True)
    o_ref[...] = (y - mean) * lax.rsqrt(var + 1e-5) * g_ref[...] + b_ref[...]


def _combine(a0, a1, x, Wo, gamma, beta, S):
    blk = 1000
    grid = (N // blk,)
    return pl.pallas_call(
        _combine_body,
        grid=grid,
        in_specs=[
            pl.BlockSpec((blk, ACCW), lambda i: (i, 0)),
            pl.BlockSpec((blk, ACCW), lambda i: (i, 0)),
            pl.BlockSpec((blk, D), lambda i: (i, 0)),
            pl.BlockSpec((D, D), lambda i: (0, 0)),
            pl.BlockSpec((1, D), lambda i: (0, 0)),
            pl.BlockSpec((1, D), lambda i: (0, 0)),
            pl.BlockSpec((4, 128), lambda i: (0, 0)),
        ],
        out_specs=pl.BlockSpec((blk, D), lambda i: (i, 0)),
        out_shape=jax.ShapeDtypeStruct((N, D), jnp.float32),
    )(a0, a1, x, Wo, gamma, beta, S)


def kernel(node_embeddings, edge_index, edge_weights, Wq, Wk, Wv, We, Wo, gamma, beta):
    x = node_embeddings[0]
    qs, kvs = _qkv(x, Wq, Wk, Wv)

    pad = EPTP - E // NTILES
    ei = edge_index.astype(jnp.int32)
    src = ei[0].reshape(NTILES, E // NTILES)
    tgt = ei[1].reshape(NTILES, E // NTILES)
    ewb = lax.bitcast_convert_type(edge_weights, jnp.int32).reshape(NTILES, -1)
    zpad = jnp.zeros((NTILES, pad), jnp.int32)
    srcg = jnp.concatenate([src, zpad], 1).reshape(NTILES, NB, BATCH)
    tgtp = jnp.concatenate([tgt, zpad], 1).reshape(NTILES, NB, BATCH)
    ewbp = jnp.concatenate([ewb, zpad], 1).reshape(NTILES, NB, BATCH)
    srcs = jnp.concatenate([src, jnp.full((NTILES, pad), N, jnp.int32)], 1)
    srcs = srcs.reshape(NTILES, NB, BATCH)
    edata = jnp.stack([srcg, tgtp, ewbp, srcs], axis=2)

    we_exp = jnp.broadcast_to(We[:, 0].reshape(2, 4, 1), (2, 4, 16)).astype(jnp.float32)

    outn = _sc_edge_call(qs, kvs, edata, we_exp)

    S = (jnp.arange(128)[None, :] // DH == jnp.arange(4)[:, None]).astype(jnp.float32)
    ln = _combine(outn[0], outn[1], x, Wo,
                  gamma.reshape(1, D), beta.reshape(1, D), S)
    return ln[None]
